# baseline (speedup 1.0000x reference)
import functools

import jax
import jax.numpy as jnp
from jax import lax
from jax.experimental import pallas as pl
from jax.experimental.pallas import tpu as pltpu
from jax.experimental.pallas import tpu_sc as plsc

N = 10000
D = 128
H = 64
G = 64

NC = 2
NS = 16
NW = NC * NS

B = 128
NBUF = 4
F0 = 48
F1 = 16
N_PAD = NS * 640
ROWS_PT = N_PAD // NS

_mesh = plsc.VectorSubcoreMesh(core_axis_name="c", subcore_axis_name="s")
_sc_params = pltpu.CompilerParams(use_tc_tiling_on_sc=False)


def _edge_layout(E):
    q = B * NBUF * NS
    e_pad = (E + q - 1) // q * q
    return e_pad, e_pad // B


@functools.lru_cache(maxsize=None)
def _make_deg_kernel(E):
    E_PAD, TOT_CHUNKS = _edge_layout(E)
    PER_W = E_PAD // NW
    CHUNKS = PER_W // B

    @functools.partial(
        pl.kernel,
        out_type=jax.ShapeDtypeStruct((NC, N_PAD, 8), jnp.float32),
        mesh=_mesh,
        scratch_types=[
            pltpu.VMEM_SHARED((N_PAD, 8), jnp.float32),
            pltpu.VMEM((CHUNKS, B), jnp.int32),
            pltpu.VMEM((B, 8), jnp.float32),
            pltpu.SemaphoreType.DMA((NBUF,)),
        ],
        compiler_params=_sc_params,
    )
    def deg_kernel(dst_hbm, zeros_hbm, ones_hbm, out_hbm, accum, dstbuf, ones_v, sems):
        cid = lax.axis_index("c")
        sid = lax.axis_index("s")
        wid = cid * NS + sid
        pltpu.sync_copy(dst_hbm.at[wid], dstbuf)
        pltpu.sync_copy(zeros_hbm.at[pl.ds(0, ROWS_PT)], accum.at[pl.ds(sid * ROWS_PT, ROWS_PT)])
        pltpu.sync_copy(ones_hbm, ones_v)
        plsc.subcore_barrier()

        def group(g, carry):
            for b in range(NBUF):
                pltpu.async_copy(ones_v, accum.at[dstbuf.at[g * NBUF + b]],
                                 sems.at[b], add=True)
            for b in range(NBUF):
                pltpu.make_async_copy(ones_v, accum.at[dstbuf.at[g * NBUF + b]],
                                      sems.at[b]).wait()
            return carry

        lax.fori_loop(0, CHUNKS // NBUF, group, 0)
        plsc.subcore_barrier()
        r0 = sid * ROWS_PT
        pltpu.sync_copy(accum.at[pl.ds(r0, ROWS_PT)], out_hbm.at[cid, pl.ds(r0, ROWS_PT)])

    return deg_kernel


@functools.lru_cache(maxsize=None)
def _make_agg_kernel(E):
    E_PAD, TOT_CHUNKS = _edge_layout(E)
    CT = TOT_CHUNKS // NS
    assert CT % NBUF == 0 and CT // NBUF >= 2

    @functools.partial(
        pl.kernel,
        out_type=(jax.ShapeDtypeStruct((N_PAD, F0), jnp.float32),
                  jax.ShapeDtypeStruct((N_PAD, F1), jnp.float32)),
        mesh=_mesh,
        scratch_types=[
            pltpu.VMEM_SHARED((N_PAD, F0), jnp.float32),
            pltpu.VMEM_SHARED((N_PAD, F1), jnp.float32),
            pltpu.VMEM((CT, B), jnp.int32),
            pltpu.VMEM((CT, B), jnp.int32),
            pltpu.VMEM((NBUF, B, F0), jnp.float32),
            pltpu.VMEM((NBUF, B, F1), jnp.float32),
            pltpu.SemaphoreType.DMA((NBUF,)),
            pltpu.SemaphoreType.DMA((NBUF,)),
        ],
        compiler_params=_sc_params,
    )
    def agg_kernel(src_hbm, dst_hbm, za_hbm, zb_hbm, zeros_a, zeros_b,
                   outa_hbm, outb_hbm,
                   accum_a, accum_b, sidx, didx, rows_a, rows_b, gsem, ssem):
        cid = lax.axis_index("c")
        sid = lax.axis_index("s")

        with jax.named_scope("idx_preload"):
            base = sid * CT
            pltpu.sync_copy(src_hbm.at[pl.ds(base, CT)], sidx)
            pltpu.sync_copy(dst_hbm.at[pl.ds(base, CT)], didx)

        def run_side(z_hbm, zeros_hbm, out_hbm, accum, rows):
            def gather(i, b):
                pltpu.async_copy(z_hbm.at[sidx.at[i]], rows.at[b], gsem.at[b])

            def gather_wait(b):
                pltpu.make_async_copy(z_hbm.at[sidx.at[0]], rows.at[b], gsem.at[b]).wait()

            def scatter(i, b):
                pltpu.async_copy(rows.at[b], accum.at[didx.at[i]], ssem.at[b], add=True)

            def scatter_wait(i, b):
                pltpu.make_async_copy(rows.at[b], accum.at[didx.at[i]], ssem.at[b]).wait()

            with jax.named_scope("zero_prime"):
                for b in range(NBUF):
                    gather(b, b)
                pltpu.sync_copy(zeros_hbm.at[pl.ds(0, ROWS_PT)],
                                accum.at[pl.ds(sid * ROWS_PT, ROWS_PT)])
                plsc.subcore_barrier()

            def group(g, carry):
                i0 = g * NBUF
                for b in range(NBUF):
                    gather_wait(b)
                    scatter(i0 + b, b)
                for b in range(NBUF):
                    scatter_wait(i0 + b, b)
                    gather(i0 + NBUF + b, b)
                return carry

            with jax.named_scope("edge_loop"):
                lax.fori_loop(0, CT // NBUF - 1, group, 0)
                i0 = CT - NBUF
                for b in range(NBUF):
                    gather_wait(b)
                    scatter(i0 + b, b)
                for b in range(NBUF):
                    scatter_wait(i0 + b, b)
            with jax.named_scope("drain_out"):
                plsc.subcore_barrier()
                r0 = sid * ROWS_PT
                pltpu.sync_copy(accum.at[pl.ds(r0, ROWS_PT)], out_hbm.at[pl.ds(r0, ROWS_PT)])

        @pl.when(cid == 0)
        def _():
            run_side(za_hbm, zeros_a, outa_hbm, accum_a, rows_a)

        @pl.when(cid == 1)
        def _():
            run_side(zb_hbm, zeros_b, outb_hbm, accum_b, rows_b)

    return agg_kernel


def _tc1_body(degA, degB, x, W1a, W1b, dis_o, z1a_o, z1b_o):
    dis = lax.rsqrt(degA[...] + degB[...] + 1.0)
    dis_o[...] = dis
    z1a_o[...] = dis * jnp.dot(x[...], W1a[...], preferred_element_type=jnp.float32)
    z1b_o[...] = dis * jnp.dot(x[...], W1b[...], preferred_element_type=jnp.float32)


def _tc2_body(pa, pb, z1a, z1b, dis, b1a, b1b, z2a_o, z2b_o):
    d = dis[...]
    z2a_o[...] = d * jnp.maximum(d * (pa[...] + z1a[...]) + b1a[...], 0.0)
    z2b_o[...] = d * jnp.maximum(d * (pb[...] + z1b[...]) + b1b[...], 0.0)


def _tc3_body(pa, pb, z2a, z2b, dis, W2a, W2b, b2,
              z3q0_o, z3q1_o, z3q2_o, z3q3_o):
    d = dis[...]
    a2a = d * (pa[...] + z2a[...])
    a2b = d * (pb[...] + z2b[...])
    h = jnp.dot(a2a, W2a[...], preferred_element_type=jnp.float32)
    h = h + jnp.dot(a2b, W2b[...], preferred_element_type=jnp.float32)
    z3 = d * jnp.maximum(h + b2[...], 0.0)
    z3q0_o[...] = z3[:, 0:F0]
    z3q1_o[...] = z3[:, F0:H]
    z3q2_o[...] = z3[:, H:H + F0]
    z3q3_o[...] = z3[:, H + F0:]


def _tc4_body(p0, p1, p2, p3, z3q0, z3q1, z3q2, z3q3, dis, batch2,
              W3, b3, Wl, bl, out_o):
    d = dis[...]
    seg = lax.broadcasted_iota(jnp.int32, (G, N), 0)
    onehot = (batch2[...] == seg).astype(jnp.float32)
    sums = jnp.concatenate(
        [jnp.dot(onehot, d * (p[...] + z[...]), preferred_element_type=jnp.float32)
         for p, z in ((p0, z3q0), (p1, z3q1), (p2, z3q2), (p3, z3q3))], axis=1)
    cnt = jnp.sum(onehot, axis=1, keepdims=True)
    pooled = sums / jnp.maximum(cnt, 1.0)
    head = jnp.dot(pooled, W3[...], preferred_element_type=jnp.float32) + b3[...]
    out_o[...] = jnp.dot(head, Wl[...], preferred_element_type=jnp.float32) + bl[...]


def _tc_call(body, out_shapes):
    return pl.pallas_call(body, out_shape=out_shapes)


def _f32(shape):
    return jax.ShapeDtypeStruct(shape, jnp.float32)


def kernel(x, edge_index, batch, W1, b1, W2, b2, W3, b3, Wl, bl):
    E = edge_index.shape[1]
    E_PAD, TOT_CHUNKS = _edge_layout(E)
    pad = E_PAD - E
    src_f = jnp.concatenate([edge_index[0], jnp.zeros((pad,), jnp.int32)])
    dst_f = jnp.concatenate([edge_index[1], jnp.full((pad,), N, jnp.int32)])
    src2d = src_f.reshape(TOT_CHUNKS, B)
    dst2d = dst_f.reshape(TOT_CHUNKS, B)
    dst_w = dst_f.reshape(NW, -1, B)

    zeros8 = jnp.zeros((ROWS_PT, 8), jnp.float32)
    ones8 = jnp.ones((B, 8), jnp.float32)
    zeros_a = jnp.zeros((ROWS_PT, F0), jnp.float32)
    zeros_b = jnp.zeros((ROWS_PT, F1), jnp.float32)

    deg_parts = _make_deg_kernel(E)(dst_w, zeros8, ones8)
    degA = deg_parts[0, :N, 0:1]
    degB = deg_parts[1, :N, 0:1]

    agg = _make_agg_kernel(E)

    dis, z1a, z1b = _tc_call(_tc1_body, (_f32((N, 1)), _f32((N, F0)), _f32((N, F1))))(
        degA, degB, x, W1[:, :F0], W1[:, F0:])

    s1a, s1b = agg(src2d, dst2d, z1a, z1b, zeros_a, zeros_b)
    z2a, z2b = _tc_call(_tc2_body, (_f32((N, F0)), _f32((N, F1))))(
        s1a[:N], s1b[:N], z1a, z1b, dis,
        b1[:F0].reshape(1, F0), b1[F0:].reshape(1, F1))

    s2a, s2b = agg(src2d, dst2d, z2a, z2b, zeros_a, zeros_b)
    z3q0, z3q1, z3q2, z3q3 = _tc_call(_tc3_body, (
        _f32((N, F0)), _f32((N, F1)), _f32((N, F0)), _f32((N, F1))))(
        s2a[:N], s2b[:N], z2a, z2b, dis, W2[:F0], W2[F0:], b2.reshape(1, 2 * H))

    s3a0, s3b0 = agg(src2d, dst2d, z3q0, z3q1, zeros_a, zeros_b)
    s3a1, s3b1 = agg(src2d, dst2d, z3q2, z3q3, zeros_a, zeros_b)

    out = _tc_call(_tc4_body, _f32((G, 1)))(
        s3a0[:N], s3b0[:N], s3a1[:N], s3b1[:N], z3q0, z3q1, z3q2, z3q3,
        dis, batch.reshape(1, N),
        W3, b3.reshape(1, 2 * H), Wl, bl.reshape(1, 1))
    return out


# speedup vs baseline: 1.2906x; 1.2906x over previous
import functools

import jax
import jax.numpy as jnp
from jax import lax
from jax.experimental import pallas as pl
from jax.experimental.pallas import tpu as pltpu
from jax.experimental.pallas import tpu_sc as plsc

N = 10000
D = 128
H = 64
G = 64

NC = 2
NS = 16
NW = NC * NS

B = 128
NBUF = 4
F0 = 32
F1 = 32
N_PAD = NS * 640
ROWS_PT = N_PAD // NS

_mesh = plsc.VectorSubcoreMesh(core_axis_name="c", subcore_axis_name="s")
_sc_params = pltpu.CompilerParams(use_tc_tiling_on_sc=False)


def _edge_layout(E):
    q = B * NBUF * NS
    e_pad = (E + q - 1) // q * q
    return e_pad, e_pad // B


@functools.lru_cache(maxsize=None)
def _make_deg_kernel(E):
    E_PAD, TOT_CHUNKS = _edge_layout(E)
    PER_W = E_PAD // NW
    CHUNKS = PER_W // B

    @functools.partial(
        pl.kernel,
        out_type=jax.ShapeDtypeStruct((NC, N_PAD, 8), jnp.float32),
        mesh=_mesh,
        scratch_types=[
            pltpu.VMEM_SHARED((N_PAD, 8), jnp.float32),
            pltpu.VMEM((CHUNKS, B), jnp.int32),
            pltpu.VMEM((B, 8), jnp.float32),
            pltpu.SemaphoreType.DMA((NBUF,)),
        ],
        compiler_params=_sc_params,
    )
    def deg_kernel(dst_hbm, zeros_hbm, ones_hbm, out_hbm, accum, dstbuf, ones_v, sems):
        cid = lax.axis_index("c")
        sid = lax.axis_index("s")
        wid = cid * NS + sid
        pltpu.sync_copy(dst_hbm.at[wid], dstbuf)
        pltpu.sync_copy(zeros_hbm.at[pl.ds(0, ROWS_PT)], accum.at[pl.ds(sid * ROWS_PT, ROWS_PT)])
        pltpu.sync_copy(ones_hbm, ones_v)
        plsc.subcore_barrier()

        def group(g, carry):
            for b in range(NBUF):
                pltpu.async_copy(ones_v, accum.at[dstbuf.at[g * NBUF + b]],
                                 sems.at[b], add=True)
            for b in range(NBUF):
                pltpu.make_async_copy(ones_v, accum.at[dstbuf.at[g * NBUF + b]],
                                      sems.at[b]).wait()
            return carry

        lax.fori_loop(0, CHUNKS // NBUF, group, 0)
        plsc.subcore_barrier()
        r0 = sid * ROWS_PT
        pltpu.sync_copy(accum.at[pl.ds(r0, ROWS_PT)], out_hbm.at[cid, pl.ds(r0, ROWS_PT)])

    return deg_kernel


@functools.lru_cache(maxsize=None)
def _make_agg_kernel(E):
    E_PAD, TOT_CHUNKS = _edge_layout(E)
    CT = TOT_CHUNKS // NS
    assert CT % NBUF == 0 and CT // NBUF >= 2

    @functools.partial(
        pl.kernel,
        out_type=(jax.ShapeDtypeStruct((N_PAD, F0), jnp.float32),
                  jax.ShapeDtypeStruct((N_PAD, F1), jnp.float32)),
        mesh=_mesh,
        scratch_types=[
            pltpu.VMEM_SHARED((N_PAD, F0), jnp.float32),
            pltpu.VMEM_SHARED((N_PAD, F1), jnp.float32),
            pltpu.VMEM((CT, B), jnp.int32),
            pltpu.VMEM((CT, B), jnp.int32),
            pltpu.VMEM((NBUF, B, F0), jnp.float32),
            pltpu.VMEM((NBUF, B, F1), jnp.float32),
            pltpu.SemaphoreType.DMA((NBUF,)),
            pltpu.SemaphoreType.DMA((NBUF,)),
        ],
        compiler_params=_sc_params,
    )
    def agg_kernel(src_hbm, dst_hbm, za_hbm, zb_hbm, zeros_a, zeros_b,
                   outa_hbm, outb_hbm,
                   accum_a, accum_b, sidx, didx, rows_a, rows_b, gsem, ssem):
        cid = lax.axis_index("c")
        sid = lax.axis_index("s")

        with jax.named_scope("idx_preload"):
            base = sid * CT
            pltpu.sync_copy(src_hbm.at[pl.ds(base, CT)], sidx)
            pltpu.sync_copy(dst_hbm.at[pl.ds(base, CT)], didx)

        def run_side(z_hbm, zeros_hbm, out_hbm, accum, rows):
            def gather(i, b):
                pltpu.async_copy(z_hbm.at[sidx.at[i]], rows.at[b], gsem.at[b])

            def gather_wait(b):
                pltpu.make_async_copy(z_hbm.at[sidx.at[0]], rows.at[b], gsem.at[b]).wait()

            def scatter(i, b):
                pltpu.async_copy(rows.at[b], accum.at[didx.at[i]], ssem.at[b], add=True)

            def scatter_wait(i, b):
                pltpu.make_async_copy(rows.at[b], accum.at[didx.at[i]], ssem.at[b]).wait()

            with jax.named_scope("zero_prime"):
                for b in range(NBUF):
                    gather(b, b)
                pltpu.sync_copy(zeros_hbm.at[pl.ds(0, ROWS_PT)],
                                accum.at[pl.ds(sid * ROWS_PT, ROWS_PT)])
                plsc.subcore_barrier()

            def group(g, carry):
                i0 = g * NBUF
                for b in range(NBUF):
                    gather_wait(b)
                    scatter(i0 + b, b)
                for b in range(NBUF):
                    scatter_wait(i0 + b, b)
                    gather(i0 + NBUF + b, b)
                return carry

            with jax.named_scope("edge_loop"):
                lax.fori_loop(0, CT // NBUF - 1, group, 0)
                i0 = CT - NBUF
                for b in range(NBUF):
                    gather_wait(b)
                    scatter(i0 + b, b)
                for b in range(NBUF):
                    scatter_wait(i0 + b, b)
            with jax.named_scope("drain_out"):
                plsc.subcore_barrier()
                r0 = sid * ROWS_PT
                pltpu.sync_copy(accum.at[pl.ds(r0, ROWS_PT)], out_hbm.at[pl.ds(r0, ROWS_PT)])

        @pl.when(cid == 0)
        def _():
            run_side(za_hbm, zeros_a, outa_hbm, accum_a, rows_a)

        @pl.when(cid == 1)
        def _():
            run_side(zb_hbm, zeros_b, outb_hbm, accum_b, rows_b)

    return agg_kernel


def _tc1_body(degA, degB, x, W1a, W1b, dis_o, z1a_o, z1b_o):
    dis = lax.rsqrt(degA[...] + degB[...] + 1.0)
    dis_o[...] = dis
    z1a_o[...] = dis * jnp.dot(x[...], W1a[...], preferred_element_type=jnp.float32)
    z1b_o[...] = dis * jnp.dot(x[...], W1b[...], preferred_element_type=jnp.float32)


def _tc2_body(pa, pb, z1a, z1b, dis, b1a, b1b, z2a_o, z2b_o):
    d = dis[...]
    z2a_o[...] = d * jnp.maximum(d * (pa[...] + z1a[...]) + b1a[...], 0.0)
    z2b_o[...] = d * jnp.maximum(d * (pb[...] + z1b[...]) + b1b[...], 0.0)


def _tc3_body(pa, pb, z2a, z2b, dis, W2a, W2b, b2,
              z3q0_o, z3q1_o, z3q2_o, z3q3_o):
    d = dis[...]
    a2a = d * (pa[...] + z2a[...])
    a2b = d * (pb[...] + z2b[...])
    h = jnp.dot(a2a, W2a[...], preferred_element_type=jnp.float32)
    h = h + jnp.dot(a2b, W2b[...], preferred_element_type=jnp.float32)
    z3 = d * jnp.maximum(h + b2[...], 0.0)
    z3q0_o[...] = z3[:, 0:F0]
    z3q1_o[...] = z3[:, F0:H]
    z3q2_o[...] = z3[:, H:H + F0]
    z3q3_o[...] = z3[:, H + F0:]


def _tc4_body(p0, p1, p2, p3, z3q0, z3q1, z3q2, z3q3, dis, batch2,
              W3, b3, Wl, bl, out_o):
    d = dis[...]
    seg = lax.broadcasted_iota(jnp.int32, (G, N), 0)
    onehot = (batch2[...] == seg).astype(jnp.float32)
    sums = jnp.concatenate(
        [jnp.dot(onehot, d * (p[...] + z[...]), preferred_element_type=jnp.float32)
         for p, z in ((p0, z3q0), (p1, z3q1), (p2, z3q2), (p3, z3q3))], axis=1)
    cnt = jnp.sum(onehot, axis=1, keepdims=True)
    pooled = sums / jnp.maximum(cnt, 1.0)
    head = jnp.dot(pooled, W3[...], preferred_element_type=jnp.float32) + b3[...]
    out_o[...] = jnp.dot(head, Wl[...], preferred_element_type=jnp.float32) + bl[...]


def _tc_call(body, out_shapes):
    return pl.pallas_call(body, out_shape=out_shapes)


def _f32(shape):
    return jax.ShapeDtypeStruct(shape, jnp.float32)


def kernel(x, edge_index, batch, W1, b1, W2, b2, W3, b3, Wl, bl):
    E = edge_index.shape[1]
    E_PAD, TOT_CHUNKS = _edge_layout(E)
    pad = E_PAD - E
    src_f = jnp.concatenate([edge_index[0], jnp.zeros((pad,), jnp.int32)])
    dst_f = jnp.concatenate([edge_index[1], jnp.full((pad,), N, jnp.int32)])
    src2d = src_f.reshape(TOT_CHUNKS, B)
    dst2d = dst_f.reshape(TOT_CHUNKS, B)
    dst_w = dst_f.reshape(NW, -1, B)

    zeros8 = jnp.zeros((ROWS_PT, 8), jnp.float32)
    ones8 = jnp.ones((B, 8), jnp.float32)
    zeros_a = jnp.zeros((ROWS_PT, F0), jnp.float32)
    zeros_b = jnp.zeros((ROWS_PT, F1), jnp.float32)

    deg_parts = _make_deg_kernel(E)(dst_w, zeros8, ones8)
    degA = deg_parts[0, :N, 0:1]
    degB = deg_parts[1, :N, 0:1]

    agg = _make_agg_kernel(E)

    dis, z1a, z1b = _tc_call(_tc1_body, (_f32((N, 1)), _f32((N, F0)), _f32((N, F1))))(
        degA, degB, x, W1[:, :F0], W1[:, F0:])

    s1a, s1b = agg(src2d, dst2d, z1a, z1b, zeros_a, zeros_b)
    z2a, z2b = _tc_call(_tc2_body, (_f32((N, F0)), _f32((N, F1))))(
        s1a[:N], s1b[:N], z1a, z1b, dis,
        b1[:F0].reshape(1, F0), b1[F0:].reshape(1, F1))

    s2a, s2b = agg(src2d, dst2d, z2a, z2b, zeros_a, zeros_b)
    z3q0, z3q1, z3q2, z3q3 = _tc_call(_tc3_body, (
        _f32((N, F0)), _f32((N, F1)), _f32((N, F0)), _f32((N, F1))))(
        s2a[:N], s2b[:N], z2a, z2b, dis, W2[:F0], W2[F0:], b2.reshape(1, 2 * H))

    s3a0, s3b0 = agg(src2d, dst2d, z3q0, z3q1, zeros_a, zeros_b)
    s3a1, s3b1 = agg(src2d, dst2d, z3q2, z3q3, zeros_a, zeros_b)

    out = _tc_call(_tc4_body, _f32((G, 1)))(
        s3a0[:N], s3b0[:N], s3a1[:N], s3b1[:N], z3q0, z3q1, z3q2, z3q3,
        dis, batch.reshape(1, N),
        W3, b3.reshape(1, 2 * H), Wl, bl.reshape(1, 1))
    return out
